# Initial kernel scaffold; baseline (speedup 1.0000x reference)
import functools

import jax
import jax.numpy as jnp
from jax.experimental import pallas as pl
from jax.experimental.pallas import tpu as pltpu

_K = 32
_NEG = -1e30


def _dfm_kernel(hr_ref, lr_ref, w1_ref, b1_ref, w2_ref, b2_ref, out_ref):
    hr = hr_ref[0]
    lr = lr_ref[0]
    w1 = w1_ref[...]
    b1 = b1_ref[0, 0]
    w2 = w2_ref[0, 0]
    b2 = b2_ref[0, 0]

    corr = jax.lax.dot_general(
        lr, hr, (((1,), (1,)), ((), ())),
        preferred_element_type=jnp.float32,
        precision=jax.lax.Precision.HIGHEST,
    )

    cols = []
    for c in range(64):
        ref_col = corr[:, c:c + 1]
        greater = (corr > ref_col).astype(jnp.float32)
        if c > 0:
            ties = (corr[:, :c] == ref_col).astype(jnp.float32)
            rank_c = jnp.sum(greater, axis=1, keepdims=True) + jnp.sum(
                ties, axis=1, keepdims=True)
        else:
            rank_c = jnp.sum(greater, axis=1, keepdims=True)
        cols.append(rank_c)
    rank = jnp.concatenate(cols, axis=1)

    a_mat = jnp.zeros((64, 64), dtype=jnp.float32)
    for r in range(_K):
        a_mat = a_mat + jnp.where(rank == float(r), w1[0, 1 + r], 0.0)
    neg_bias = jnp.where(rank < float(_K), 0.0, _NEG)

    wsum = jax.lax.dot_general(
        a_mat, hr, (((1,), (1,)), ((), ())),
        preferred_element_type=jnp.float32,
        precision=jax.lax.Precision.HIGHEST,
    )

    m = jnp.full((64, 4096), _NEG, dtype=jnp.float32)
    for c in range(64):
        cand = hr[c:c + 1, :] + neg_bias[:, c:c + 1]
        m = jnp.maximum(m, cand)

    y = w1[0, 0] * lr + wsum + b1
    y = jnp.where(y >= 0.0, y, 0.1 * y)
    y = w2 * y + b2
    out_ref[0] = y * (1.0 + jax.nn.sigmoid(m))


@jax.jit
def kernel(HR, LR, w1, b1, w2, b2):
    B, C, H, W = HR.shape
    hw = H * W
    hr_flat = HR.reshape(B, C, hw)
    lr_flat = LR.reshape(B, C, hw)
    w1r = w1.reshape(1, 1 + _K)
    b1r = b1.reshape(1, 1)
    w2r = w2.reshape(1, 1)
    b2r = b2.reshape(1, 1)

    out = pl.pallas_call(
        _dfm_kernel,
        grid=(B,),
        in_specs=[
            pl.BlockSpec((1, C, hw), lambda b: (b, 0, 0)),
            pl.BlockSpec((1, C, hw), lambda b: (b, 0, 0)),
            pl.BlockSpec((1, 1 + _K), lambda b: (0, 0)),
            pl.BlockSpec((1, 1), lambda b: (0, 0)),
            pl.BlockSpec((1, 1), lambda b: (0, 0)),
            pl.BlockSpec((1, 1), lambda b: (0, 0)),
        ],
        out_specs=pl.BlockSpec((1, C, hw), lambda b: (b, 0, 0)),
        out_shape=jax.ShapeDtypeStruct((B, C, hw), jnp.float32),
        compiler_params=pltpu.CompilerParams(
            dimension_semantics=("arbitrary",),
        ),
    )(hr_flat, lr_flat, w1r, b1r, w2r, b2r)
    return out.reshape(B, C, H, W)


# speedup vs baseline: 6.3120x; 6.3120x over previous
import functools

import jax
import jax.numpy as jnp
from jax.experimental import pallas as pl
from jax.experimental.pallas import tpu as pltpu

_K = 32
_NEG = -1e30


def _dfm_kernel(hr_ref, lr_ref, w1_ref, b1_ref, w2_ref, b2_ref, out_ref):
    hr = hr_ref[0]
    lr = lr_ref[0]
    w1 = w1_ref[...]
    b1 = b1_ref[0, 0]
    w2 = w2_ref[0, 0]
    b2 = b2_ref[0, 0]

    corr = jax.lax.dot_general(
        lr.astype(jnp.bfloat16), hr.astype(jnp.bfloat16),
        (((1,), (1,)), ((), ())),
        preferred_element_type=jnp.float32,
    )

    cols = []
    for c in range(64):
        ref_col = corr[:, c:c + 1]
        greater = (corr > ref_col).astype(jnp.float32)
        if c > 0:
            ties = (corr[:, :c] == ref_col).astype(jnp.float32)
            rank_c = jnp.sum(greater, axis=1, keepdims=True) + jnp.sum(
                ties, axis=1, keepdims=True)
        else:
            rank_c = jnp.sum(greater, axis=1, keepdims=True)
        cols.append(rank_c)
    rank = jnp.concatenate(cols, axis=1)

    a_mat = jnp.zeros((64, 64), dtype=jnp.float32)
    for r in range(_K):
        a_mat = a_mat + jnp.where(rank == float(r), w1[0, 1 + r], 0.0)
    neg_bias = jnp.where(rank < float(_K), 0.0, _NEG)

    wsum = jax.lax.dot_general(
        a_mat.astype(jnp.bfloat16), hr.astype(jnp.bfloat16),
        (((1,), (0,)), ((), ())),
        preferred_element_type=jnp.float32,
    )

    m = jnp.full((64, 4096), _NEG, dtype=jnp.float32)
    for c in range(64):
        cand = hr[c:c + 1, :] + neg_bias[:, c:c + 1]
        m = jnp.maximum(m, cand)

    w1_0 = w1[0, 0].astype(jnp.bfloat16).astype(jnp.float32)
    lr_bf = lr.astype(jnp.bfloat16).astype(jnp.float32)
    y = w1_0 * lr_bf + wsum + b1
    y = jnp.where(y >= 0.0, y, 0.1 * y)
    y = w2 * y + b2
    out_ref[0] = y * (1.0 + jax.nn.sigmoid(m))


@jax.jit
def kernel(HR, LR, w1, b1, w2, b2):
    B, C, H, W = HR.shape
    hw = H * W
    hr_flat = HR.reshape(B, C, hw)
    lr_flat = LR.reshape(B, C, hw)
    w1r = w1.reshape(1, 1 + _K)
    b1r = b1.reshape(1, 1)
    w2r = w2.reshape(1, 1)
    b2r = b2.reshape(1, 1)

    out = pl.pallas_call(
        _dfm_kernel,
        grid=(B,),
        in_specs=[
            pl.BlockSpec((1, C, hw), lambda b: (b, 0, 0)),
            pl.BlockSpec((1, C, hw), lambda b: (b, 0, 0)),
            pl.BlockSpec((1, 1 + _K), lambda b: (0, 0)),
            pl.BlockSpec((1, 1), lambda b: (0, 0)),
            pl.BlockSpec((1, 1), lambda b: (0, 0)),
            pl.BlockSpec((1, 1), lambda b: (0, 0)),
        ],
        out_specs=pl.BlockSpec((1, C, hw), lambda b: (b, 0, 0)),
        out_shape=jax.ShapeDtypeStruct((B, C, hw), jnp.float32),
        compiler_params=pltpu.CompilerParams(
            dimension_semantics=("arbitrary",),
        ),
    )(hr_flat, lr_flat, w1r, b1r, w2r, b2r)
    return out.reshape(B, C, H, W)


# speedup vs baseline: 8.7037x; 1.3789x over previous
import functools

import jax
import jax.numpy as jnp
from jax.experimental import pallas as pl
from jax.experimental.pallas import tpu as pltpu

_K = 32
_NEG = -1e30


def _dfm_kernel(hr_ref, lr_ref, w1_ref, b1_ref, w2_ref, b2_ref, out_ref):
    hr = hr_ref[0]
    lr = lr_ref[0]
    w1 = w1_ref[...]
    b1 = b1_ref[0, 0]
    w2 = w2_ref[0, 0]
    b2 = b2_ref[0, 0]

    corr = jax.lax.dot_general(
        lr.astype(jnp.bfloat16), hr.astype(jnp.bfloat16),
        (((1,), (1,)), ((), ())),
        preferred_element_type=jnp.float32,
    )

    lane = jax.lax.broadcasted_iota(jnp.int32, (64, 64), 1)
    r0 = jnp.zeros((64, 64), dtype=jnp.float32)
    r1 = r0
    r2 = r0
    r3 = r0
    for cp in range(0, 64, 4):
        def step(cpi):
            col = corr[:, cpi:cpi + 1]
            gt = col > corr
            tie = (col == corr) & (lane > cpi)
            return jnp.where(gt | tie, 1.0, 0.0)
        r0 = r0 + step(cp)
        r1 = r1 + step(cp + 1)
        r2 = r2 + step(cp + 2)
        r3 = r3 + step(cp + 3)
    rank = (r0 + r1) + (r2 + r3)

    a_mat = jnp.zeros((64, 64), dtype=jnp.float32)
    for r in range(_K):
        a_mat = a_mat + jnp.where(rank == float(r), w1[0, 1 + r], 0.0)
    neg_bias = jnp.where(rank < float(_K), 0.0, _NEG)

    wsum = jax.lax.dot_general(
        a_mat.astype(jnp.bfloat16), hr.astype(jnp.bfloat16),
        (((1,), (0,)), ((), ())),
        preferred_element_type=jnp.float32,
    )

    w1_0 = w1[0, 0].astype(jnp.bfloat16).astype(jnp.float32)
    _W = 256
    for ch in range(0, 4096, _W):
        mc0 = jnp.full((64, _W), _NEG, dtype=jnp.float32)
        mc1 = mc0
        mc2 = mc0
        mc3 = mc0
        for c in range(0, 64, 4):
            mc0 = jnp.maximum(mc0, hr[c:c + 1, ch:ch + _W] + neg_bias[:, c:c + 1])
            mc1 = jnp.maximum(mc1, hr[c + 1:c + 2, ch:ch + _W] + neg_bias[:, c + 1:c + 2])
            mc2 = jnp.maximum(mc2, hr[c + 2:c + 3, ch:ch + _W] + neg_bias[:, c + 2:c + 3])
            mc3 = jnp.maximum(mc3, hr[c + 3:c + 4, ch:ch + _W] + neg_bias[:, c + 3:c + 4])
        mc = jnp.maximum(jnp.maximum(mc0, mc1), jnp.maximum(mc2, mc3))
        lr_bf = lr[:, ch:ch + _W].astype(jnp.bfloat16).astype(jnp.float32)
        y = w1_0 * lr_bf + wsum[:, ch:ch + _W] + b1
        y = jnp.where(y >= 0.0, y, 0.1 * y)
        y = w2 * y + b2
        out_ref[0, :, ch:ch + _W] = y * (1.0 + jax.nn.sigmoid(mc))


@jax.jit
def kernel(HR, LR, w1, b1, w2, b2):
    B, C, H, W = HR.shape
    hw = H * W
    hr_flat = HR.reshape(B, C, hw)
    lr_flat = LR.reshape(B, C, hw)
    w1r = w1.reshape(1, 1 + _K)
    b1r = b1.reshape(1, 1)
    w2r = w2.reshape(1, 1)
    b2r = b2.reshape(1, 1)

    out = pl.pallas_call(
        _dfm_kernel,
        grid=(B,),
        in_specs=[
            pl.BlockSpec((1, C, hw), lambda b: (b, 0, 0)),
            pl.BlockSpec((1, C, hw), lambda b: (b, 0, 0)),
            pl.BlockSpec((1, 1 + _K), lambda b: (0, 0)),
            pl.BlockSpec((1, 1), lambda b: (0, 0)),
            pl.BlockSpec((1, 1), lambda b: (0, 0)),
            pl.BlockSpec((1, 1), lambda b: (0, 0)),
        ],
        out_specs=pl.BlockSpec((1, C, hw), lambda b: (b, 0, 0)),
        out_shape=jax.ShapeDtypeStruct((B, C, hw), jnp.float32),
        compiler_params=pltpu.CompilerParams(
            dimension_semantics=("arbitrary",),
        ),
    )(hr_flat, lr_flat, w1r, b1r, w2r, b2r)
    return out.reshape(B, C, H, W)
